# baseline (speedup 1.0000x reference)
import jax
import jax.numpy as jnp
from jax.experimental import pallas as pl
from jax.experimental.pallas import tpu as pltpu

_TOPK = 100
_NEG = -3.0e38
_BIG = 1 << 30
_BS = 8


def _body_a(inp_ref, s_ref, m_ref):
    cls = jax.nn.sigmoid(inp_ref[0, 4:, :, :])

    def shift_w(a, s):
        pad = jnp.full(a.shape[:-1] + (abs(s),), _NEG, a.dtype)
        if s > 0:
            return jnp.concatenate([a[..., s:], pad], axis=-1)
        return jnp.concatenate([pad, a[..., :s]], axis=-1)

    def shift_c(a, s):
        pad = jnp.full((abs(s),) + a.shape[1:], _NEG, a.dtype)
        if s > 0:
            return jnp.concatenate([a[s:], pad], axis=0)
        return jnp.concatenate([pad, a[:s]], axis=0)

    mw = cls
    for s in (-2, -1, 1, 2):
        mw = jnp.maximum(mw, shift_w(cls, s))
    m = mw
    for s in (-2, -1, 1, 2):
        m = jnp.maximum(m, shift_c(mw, s))
    s_val = jnp.where(m == cls, cls, 0.0)
    s_ref[0] = s_val
    m_ref[0] = jnp.max(s_val, axis=2)


def _body_b(*refs):
    s_refs = refs[0:_BS]
    m_ref, xywh_ref, out_ref, oh_ref, ow_ref, vc_ref = refs[_BS:]
    oh_ref[...] = jnp.zeros((_BS, 128, 128), jnp.float32)
    ow_ref[...] = jnp.zeros((_BS, 128, 128), jnp.float32)
    vc_ref[...] = jnp.zeros((_BS, 128, 128), jnp.float32)

    ci = jax.lax.broadcasted_iota(jnp.int32, (80, 128), 0)
    hi = jax.lax.broadcasted_iota(jnp.int32, (80, 128), 1)
    bix = ci * 128 + hi
    i128 = jax.lax.broadcasted_iota(jnp.int32, (1, 128), 1)

    m0 = tuple(m_ref[b] for b in range(_BS))

    def step(k, ms):
        out = []
        for b in range(_BS):
            mb = ms[b]
            vmax = jnp.max(mb)
            bidx = jnp.min(jnp.where(mb == vmax, bix, _BIG))
            c0 = bidx // 128
            h0 = bidx - c0 * 128
            row = s_refs[b][0, c0, pl.ds(h0, 1), :]
            w0 = jnp.min(jnp.where(row == vmax, i128, _BIG))
            oh_ref[b, pl.ds(k, 1), :] = (i128 == h0).astype(jnp.float32)
            ow_ref[b, pl.ds(k, 1), :] = (i128 == w0).astype(jnp.float32)
            vc_ref[b, pl.ds(k, 1), :] = (
                (i128 == (4 + c0)).astype(jnp.float32) * (vmax + 1.0))
            nrow = jnp.where(i128 == w0, -1.0, row)
            s_refs[b][0, c0, pl.ds(h0, 1), :] = nrow
            out.append(jnp.where(bix == bidx, jnp.max(nrow), mb))
        return tuple(out)

    jax.lax.fori_loop(0, _TOPK, step, m0)

    gw = jax.lax.broadcasted_iota(jnp.int32, (1, 128, 128), 2).astype(
        jnp.float32)
    gh = jax.lax.broadcasted_iota(jnp.int32, (1, 128, 128), 1).astype(
        jnp.float32)
    for b in range(_BS):
        bx = jax.nn.sigmoid(xywh_ref[b, 0:1, :, :]) + gw
        by = jax.nn.sigmoid(xywh_ref[b, 1:2, :, :]) + gh
        bw = jnp.exp(jnp.minimum(xywh_ref[b, 2:3, :, :], 60.0)) * 8.0
        bh = jnp.exp(jnp.minimum(xywh_ref[b, 3:4, :, :], 60.0)) * 8.0
        feats = jnp.concatenate([bx, by, bw, bh, s_refs[b][0]], axis=0)
        f2 = feats.reshape(84 * 128, 128)
        oh = oh_ref[b]
        ow = ow_ref[b]
        owt = jnp.transpose(ow)
        a = jnp.dot(f2, owt, preferred_element_type=jnp.float32)
        a3 = a.reshape(84, 128, 128)
        oht = jnp.transpose(oh)
        bm = jnp.sum(a3 * oht[None, :, :], axis=1)
        bt = jnp.transpose(bm)
        e = (jnp.dot(oh, oht, preferred_element_type=jnp.float32)
             * jnp.dot(ow, owt, preferred_element_type=jnp.float32))
        corr = jnp.dot(e, vc_ref[b, :, 0:84],
                       preferred_element_type=jnp.float32)
        btf = bt + corr
        out_ref[b, :, 0:4] = btf[0:_TOPK, 0:4] * 4.0
        out_ref[b, :, 4:5] = jnp.ones((_TOPK, 1), jnp.float32)
        out_ref[b, :, 5:85] = btf[0:_TOPK, 4:84]


def kernel(input):
    bs = input.shape[0]
    s, m = pl.pallas_call(
        _body_a,
        grid=(bs,),
        in_specs=[pl.BlockSpec((1, 84, 128, 128), lambda b: (b, 0, 0, 0))],
        out_specs=[
            pl.BlockSpec((1, 80, 128, 128), lambda b: (b, 0, 0, 0)),
            pl.BlockSpec((1, 80, 128), lambda b: (b, 0, 0)),
        ],
        out_shape=[
            jax.ShapeDtypeStruct((bs, 80, 128, 128), jnp.float32),
            jax.ShapeDtypeStruct((bs, 80, 128), jnp.float32),
        ],
    )(input)
    xywh = input[:, 0:4]
    return pl.pallas_call(
        _body_b,
        grid=(1,),
        in_specs=(
            [pl.BlockSpec((1, 80, 128, 128),
                          (lambda bb: (lambda i: (bb, 0, 0, 0)))(b))
             for b in range(bs)]
            + [pl.BlockSpec((bs, 80, 128), lambda i: (0, 0, 0)),
               pl.BlockSpec((bs, 4, 128, 128), lambda i: (0, 0, 0, 0))]
        ),
        out_specs=pl.BlockSpec((bs, _TOPK, 85), lambda i: (0, 0, 0)),
        out_shape=jax.ShapeDtypeStruct((bs, _TOPK, 85), jnp.float32),
        scratch_shapes=[
            pltpu.VMEM((_BS, 128, 128), jnp.float32),
            pltpu.VMEM((_BS, 128, 128), jnp.float32),
            pltpu.VMEM((_BS, 128, 128), jnp.float32),
        ],
    )(*([s] * bs), m, xywh)
